# Initial kernel scaffold; baseline (speedup 1.0000x reference)
import jax
import jax.numpy as jnp
from jax.experimental import pallas as pl


def kernel(boxes, scores, idxs):
    raise NotImplementedError("write your pallas kernel here")


# speedup vs baseline: 1134.2577x; 1134.2577x over previous
import jax
import jax.numpy as jnp
from jax.experimental import pallas as pl

INP_DIM = 416.0
NMS_THRES = 0.3
TOP_K = 8
N = 5000
ROWS = 40
LANES = 128
NP = ROWS * LANES


def _nms_kernel(data_ref, out_ref):
    cx = data_ref[0]
    cy = data_ref[1]
    w = data_ref[2]
    h = data_ref[3]
    s_in = data_ref[4]
    cf = data_ref[5]

    r_iota = jax.lax.broadcasted_iota(jnp.int32, (ROWS, LANES), 0)
    c_iota = jax.lax.broadcasted_iota(jnp.int32, (ROWS, LANES), 1)
    flat = r_iota * LANES + c_iota
    is_pad = flat >= N

    ninf = jnp.float32(-jnp.inf)
    s = jnp.where(is_pad, ninf, s_in)

    x1c = jnp.clip(cx - w * 0.5, 0.0, INP_DIM)
    y1c = jnp.clip(cy - h * 0.5, 0.0, INP_DIM)
    x2c = jnp.clip(cx + w * 0.5, 0.0, INP_DIM)
    y2c = jnp.clip(cy + h * 0.5, 0.0, INP_DIM)
    off = cf * (INP_DIM + 2.0)
    x1 = x1c + off
    y1 = y1c + off
    x2 = x2c + off
    y2 = y2c + off
    area = (x2 - x1 + 1.0) * (y2 - y1 + 1.0)

    alive = ~is_pad
    emitted = is_pad
    rows = []
    for _ in range(TOP_K):
        av = jnp.where(alive, s, ninf)
        m = jnp.max(av)
        valid = m > ninf
        i_main = jnp.min(jnp.where(av == m, flat, NP))
        fv = jnp.where(emitted, ninf, s)
        i_fb = jnp.min(jnp.where(fv == jnp.max(fv), flat, NP))
        i = jnp.where(valid, i_main, i_fb)
        onehot = flat == i

        def g(a):
            return jnp.sum(jnp.where(onehot, a, 0.0))

        sx1, sy1, sx2, sy2 = g(x1), g(y1), g(x2), g(y2)
        sarea = g(area)
        ix1 = jnp.maximum(x1, sx1)
        iy1 = jnp.maximum(y1, sy1)
        ix2 = jnp.minimum(x2, sx2)
        iy2 = jnp.minimum(y2, sy2)
        inter = (jnp.clip(ix2 - ix1 + 1.0, 0.0)
                 * jnp.clip(iy2 - iy1 + 1.0, 0.0))
        iou = inter / (area + sarea - inter + 1e-16)
        sup = (iou > NMS_THRES) & valid
        alive = alive & ~sup & ~onehot
        emitted = emitted | onehot

        val = jnp.where(valid, m, ninf)
        vals = (g(x1c), g(y1c), g(x2c), g(y2c), val, g(cf))
        row = jnp.zeros((1, LANES), jnp.float32)
        lane = jax.lax.broadcasted_iota(jnp.int32, (1, LANES), 1)
        for k, v in enumerate(vals):
            row = jnp.where(lane == k, v, row)
        rows.append(row)

    out_ref[...] = jnp.concatenate(rows, axis=0)


def kernel(boxes, scores, idxs):
    pad = NP - N
    def prep(a):
        return jnp.pad(a, (0, pad)).reshape(ROWS, LANES)

    data = jnp.stack([
        prep(boxes[:, 0]),
        prep(boxes[:, 1]),
        prep(boxes[:, 2]),
        prep(boxes[:, 3]),
        prep(scores),
        prep(idxs.astype(jnp.float32)),
    ])
    out = pl.pallas_call(
        _nms_kernel,
        out_shape=jax.ShapeDtypeStruct((TOP_K, LANES), jnp.float32),
    )(data)
    return out[:, :6]
